# Initial kernel scaffold; baseline (speedup 1.0000x reference)
import jax
import jax.numpy as jnp
from jax.experimental import pallas as pl


def kernel(features, edge_index, W1, b1, W2, b2, W_ih, b_ih, W_hh, b_hh, W_fc, b_fc):
    raise NotImplementedError("write your pallas kernel here")


# speedup vs baseline: 20.6838x; 20.6838x over previous
import functools

import jax
import jax.numpy as jnp
from jax import lax
from jax.experimental import pallas as pl
from jax.experimental.pallas import tpu as pltpu
from jax.experimental.pallas import tpu_sc as plsc

N = 100000
E = 3200000
IN_F = 3
H = 16

NC = 2
NS = 16
NW = NC * NS

N_ACC = 100352
R_T = N_ACC // NS
ZR = 392

CH = 128
SUP = 56
NSUP = 14
CH_PER_TILE = SUP * NSUP
NCHUNK = NW * CH_PER_TILE
E_PAD = NCHUNK * CH

BN = 2048
NB = N_ACC // BN

_mesh = plsc.VectorSubcoreMesh(core_axis_name="c", subcore_axis_name="s",
                               num_cores=NC, num_subcores=NS)


@functools.partial(
    pl.kernel,
    out_type=[jax.ShapeDtypeStruct((NC, N_ACC), jnp.float32),
              jax.ShapeDtypeStruct((NC, N_ACC), jnp.float32)],
    mesh=_mesh,
    scratch_types=[
        pltpu.VMEM((SUP, CH), jnp.int32),
        pltpu.VMEM((SUP, CH), jnp.int32),
        pltpu.VMEM((CH,), jnp.float32),
        pltpu.VMEM((R_T,), jnp.float32),
        pltpu.VMEM_SHARED((N_ACC,), jnp.float32),
        pltpu.VMEM_SHARED((N_ACC,), jnp.float32),
    ],
)
def _deg_call(src_hbm, dst_hbm, out_o, out_i, sbuf, dbuf, ones_v, zb,
              acc_o, acc_i):
    cid = lax.axis_index("c")
    sid = lax.axis_index("s")

    def init_ones(i, _):
        ones_v[pl.ds(i * 16, 16)] = jnp.ones((16,), jnp.float32)
        return 0
    lax.fori_loop(0, CH // 16, init_ones, 0)

    def init_z(i, _):
        zb[pl.ds(i * 16, 16)] = jnp.zeros((16,), jnp.float32)
        return 0
    lax.fori_loop(0, R_T // 16, init_z, 0)

    pltpu.sync_copy(zb, acc_o.at[pl.ds(sid * R_T, R_T)])
    pltpu.sync_copy(zb, acc_i.at[pl.ds(sid * R_T, R_T)])
    plsc.subcore_barrier()

    base_chunk = (cid * NS + sid) * CH_PER_TILE

    def super_body(s, _):
        pltpu.sync_copy(src_hbm.at[pl.ds(base_chunk + s * SUP, SUP)], sbuf)
        pltpu.sync_copy(dst_hbm.at[pl.ds(base_chunk + s * SUP, SUP)], dbuf)

        def chunk_body(j, _):
            pltpu.sync_copy(ones_v, acc_o.at[sbuf.at[j]], add=True)
            pltpu.sync_copy(ones_v, acc_i.at[dbuf.at[j]], add=True)
            return 0
        lax.fori_loop(0, SUP, chunk_body, 0)
        return 0
    lax.fori_loop(0, NSUP, super_body, 0)

    plsc.subcore_barrier()
    pltpu.sync_copy(acc_o.at[pl.ds(sid * R_T, R_T)],
                    out_o.at[cid, pl.ds(sid * R_T, R_T)])
    pltpu.sync_copy(acc_i.at[pl.ds(sid * R_T, R_T)],
                    out_i.at[cid, pl.ds(sid * R_T, R_T)])


@functools.partial(
    pl.kernel,
    out_type=jax.ShapeDtypeStruct((NC, N_ACC, H), jnp.float32),
    mesh=_mesh,
    compiler_params=pltpu.CompilerParams(use_tc_tiling_on_sc=False),
    scratch_types=[
        pltpu.VMEM((SUP, CH), jnp.int32),
        pltpu.VMEM((SUP, CH), jnp.int32),
        pltpu.VMEM((CH, H), jnp.float32),
        pltpu.VMEM((ZR, H), jnp.float32),
        pltpu.VMEM_SHARED((N_ACC, H), jnp.float32),
        pltpu.SemaphoreType.DMA,
    ],
)
def _agg_call(src_hbm, dst_hbm, tab_hbm, out, sbuf, dbuf, rows_v, zb,
              acc, sem):
    cid = lax.axis_index("c")
    sid = lax.axis_index("s")

    def init_z(i, _):
        zb[i] = jnp.zeros((H,), jnp.float32)
        return 0
    lax.fori_loop(0, ZR, init_z, 0)

    def zero_acc(k, _):
        pltpu.sync_copy(zb, acc.at[pl.ds(sid * R_T + k * ZR, ZR)])
        return 0
    lax.fori_loop(0, R_T // ZR, zero_acc, 0)
    plsc.subcore_barrier()

    base_chunk = (cid * NS + sid) * CH_PER_TILE

    def super_body(s, _):
        pltpu.sync_copy(src_hbm.at[pl.ds(base_chunk + s * SUP, SUP)], sbuf)
        pltpu.sync_copy(dst_hbm.at[pl.ds(base_chunk + s * SUP, SUP)], dbuf)

        def chunk_body(j, _):
            pltpu.async_copy(tab_hbm.at[sbuf.at[j]], rows_v, sem).wait()
            pltpu.sync_copy(rows_v, acc.at[dbuf.at[j]], add=True)
            return 0
        lax.fori_loop(0, SUP, chunk_body, 0)
        return 0
    lax.fori_loop(0, NSUP, super_body, 0)

    plsc.subcore_barrier()
    pltpu.sync_copy(acc.at[pl.ds(sid * R_T, R_T)],
                    out.at[cid, pl.ds(sid * R_T, R_T)])


def _prep_body(feat, dego, degi, w1, tab, dout, din):
    do = lax.rsqrt(jnp.maximum(dego[:, 0:1] + dego[:, 1:2], 1.0))
    di = lax.rsqrt(jnp.maximum(degi[:, 0:1] + degi[:, 1:2], 1.0))
    dout[...] = do
    din[...] = di
    tab[...] = jnp.dot(feat[...] * do, w1[...],
                       preferred_element_type=jnp.float32)


def _prep_call(feat, degoT, degiT, W1):
    return pl.pallas_call(
        _prep_body,
        grid=(NB,),
        in_specs=[
            pl.BlockSpec((BN, IN_F), lambda i: (i, 0)),
            pl.BlockSpec((BN, NC), lambda i: (i, 0)),
            pl.BlockSpec((BN, NC), lambda i: (i, 0)),
            pl.BlockSpec((IN_F, H), lambda i: (0, 0)),
        ],
        out_specs=[
            pl.BlockSpec((BN, H), lambda i: (i, 0)),
            pl.BlockSpec((BN, 1), lambda i: (i, 0)),
            pl.BlockSpec((BN, 1), lambda i: (i, 0)),
        ],
        out_shape=[
            jax.ShapeDtypeStruct((N_ACC, H), jnp.float32),
            jax.ShapeDtypeStruct((N_ACC, 1), jnp.float32),
            jax.ShapeDtypeStruct((N_ACC, 1), jnp.float32),
        ],
    )(feat, degoT, degiT, W1)


def _mid_body(agg, dout, din, w2, b1, tab2):
    h1 = jnp.maximum((agg[0] + agg[1]) * din[...] + b1[...], 0.0)
    tab2[...] = jnp.dot(h1 * dout[...], w2[...],
                        preferred_element_type=jnp.float32)


def _mid_call(agg1, dout, din, W2, b1):
    return pl.pallas_call(
        _mid_body,
        grid=(NB,),
        in_specs=[
            pl.BlockSpec((NC, BN, H), lambda i: (0, i, 0)),
            pl.BlockSpec((BN, 1), lambda i: (i, 0)),
            pl.BlockSpec((BN, 1), lambda i: (i, 0)),
            pl.BlockSpec((H, H), lambda i: (0, 0)),
            pl.BlockSpec((1, H), lambda i: (0, 0)),
        ],
        out_specs=pl.BlockSpec((BN, H), lambda i: (i, 0)),
        out_shape=jax.ShapeDtypeStruct((N_ACC, H), jnp.float32),
    )(agg1, dout, din, W2, b1)


def _fin_body(agg, din, b2, wr, wz, wn, br, bz, bn1, bhn, wfc, bfc, out):
    h2 = jnp.maximum((agg[0] + agg[1]) * din[...] + b2[...], 0.0)
    r = jax.nn.sigmoid(jnp.dot(h2, wr[...],
                               preferred_element_type=jnp.float32) + br[...])
    z = jax.nn.sigmoid(jnp.dot(h2, wz[...],
                               preferred_element_type=jnp.float32) + bz[...])
    n = jnp.tanh(jnp.dot(h2, wn[...], preferred_element_type=jnp.float32)
                 + bn1[...] + r * bhn[...])
    hy = (1.0 - z) * n
    out[...] = jnp.dot(hy, wfc[...],
                       preferred_element_type=jnp.float32) + bfc[...]


def _fin_call(agg2, din, b2, wr, wz, wn, br, bz, bn1, bhn, wfc, bfc):
    small = lambda r, c: pl.BlockSpec((r, c), lambda i: (0, 0))
    return pl.pallas_call(
        _fin_body,
        grid=(NB,),
        in_specs=[
            pl.BlockSpec((NC, BN, H), lambda i: (0, i, 0)),
            pl.BlockSpec((BN, 1), lambda i: (i, 0)),
            small(1, H), small(H, H), small(H, H), small(H, H),
            small(1, H), small(1, H), small(1, H), small(1, H),
            small(H, 1), small(1, 1),
        ],
        out_specs=pl.BlockSpec((BN, 1), lambda i: (i, 0)),
        out_shape=jax.ShapeDtypeStruct((N_ACC, 1), jnp.float32),
    )(agg2, din, b2, wr, wz, wn, br, bz, bn1, bhn, wfc, bfc)


def kernel(features, edge_index, W1, b1, W2, b2, W_ih, b_ih, W_hh, b_hh,
           W_fc, b_fc):
    src = edge_index[0]
    dst = edge_index[1]
    npad = E_PAD - E
    i = jnp.arange(npad, dtype=jnp.int32)
    pad_src = N + (i % (N_ACC - N))
    pad_dst = N + ((i * 7 + 3) % (N_ACC - N))
    src_p = jnp.concatenate([src, pad_src]).reshape(NCHUNK, CH)
    dst_p = jnp.concatenate([dst, pad_dst]).reshape(NCHUNK, CH)

    dego, degi = _deg_call(src_p, dst_p)

    featp = jnp.zeros((N_ACC, IN_F), jnp.float32).at[:N].set(features)
    tab1, dout, din = _prep_call(featp, dego.T, degi.T, W1)

    agg1 = _agg_call(src_p, dst_p, tab1)
    tab2 = _mid_call(agg1, dout, din, W2, b1.reshape(1, H))

    agg2 = _agg_call(src_p, dst_p, tab2)

    out = _fin_call(
        agg2, din, b2.reshape(1, H),
        W_ih[:H].T, W_ih[H:2 * H].T, W_ih[2 * H:].T,
        (b_ih[:H] + b_hh[:H]).reshape(1, H),
        (b_ih[H:2 * H] + b_hh[H:2 * H]).reshape(1, H),
        b_ih[2 * H:].reshape(1, H), b_hh[2 * H:].reshape(1, H),
        W_fc.T, b_fc.reshape(1, 1),
    )
    return out[:N]


# speedup vs baseline: 25.3839x; 1.2272x over previous
import functools

import jax
import jax.numpy as jnp
from jax import lax
from jax.experimental import pallas as pl
from jax.experimental.pallas import tpu as pltpu
from jax.experimental.pallas import tpu_sc as plsc

N = 100000
E = 3200000
IN_F = 3
H = 16

NC = 2
NS = 16
NW = NC * NS

N_ACC = 100352
R_T = N_ACC // NS
ZR = 392

CH = 128
SUP = 56
NSUP = 14
CH_PER_TILE = SUP * NSUP
NCHUNK = NW * CH_PER_TILE
E_PAD = NCHUNK * CH

NP8 = N_ACC // 8
GQ = N_ACC // 128
NB = 16
BP = NP8 // NB
BG = GQ // NB

_mesh = plsc.VectorSubcoreMesh(core_axis_name="c", subcore_axis_name="s",
                               num_cores=NC, num_subcores=NS)
_sc_params = pltpu.CompilerParams(use_tc_tiling_on_sc=False)


@functools.partial(
    pl.kernel,
    out_type=[jax.ShapeDtypeStruct((NC, N_ACC), jnp.float32),
              jax.ShapeDtypeStruct((NC, N_ACC), jnp.float32)],
    mesh=_mesh,
    compiler_params=_sc_params,
    scratch_types=[
        pltpu.VMEM((SUP, CH), jnp.int32),
        pltpu.VMEM((SUP, CH), jnp.int32),
        pltpu.VMEM((CH,), jnp.float32),
        pltpu.VMEM((R_T,), jnp.float32),
        pltpu.VMEM_SHARED((N_ACC,), jnp.float32),
        pltpu.VMEM_SHARED((N_ACC,), jnp.float32),
    ],
)
def _deg_call(src_hbm, dst_hbm, out_o, out_i, sbuf, dbuf, ones_v, zb,
              acc_o, acc_i):
    cid = lax.axis_index("c")
    sid = lax.axis_index("s")

    def init_ones(i, _):
        ones_v[pl.ds(i * 16, 16)] = jnp.ones((16,), jnp.float32)
        return 0
    lax.fori_loop(0, CH // 16, init_ones, 0)

    def init_z(i, _):
        zb[pl.ds(i * 16, 16)] = jnp.zeros((16,), jnp.float32)
        return 0
    lax.fori_loop(0, R_T // 16, init_z, 0)

    pltpu.sync_copy(zb, acc_o.at[pl.ds(sid * R_T, R_T)])
    pltpu.sync_copy(zb, acc_i.at[pl.ds(sid * R_T, R_T)])
    plsc.subcore_barrier()

    base_chunk = (cid * NS + sid) * CH_PER_TILE

    def super_body(s, _):
        pltpu.sync_copy(src_hbm.at[pl.ds(base_chunk + s * SUP, SUP)], sbuf)
        pltpu.sync_copy(dst_hbm.at[pl.ds(base_chunk + s * SUP, SUP)], dbuf)

        def chunk_body(j, _):
            pltpu.sync_copy(ones_v, acc_o.at[sbuf.at[j]], add=True)
            pltpu.sync_copy(ones_v, acc_i.at[dbuf.at[j]], add=True)
            return 0
        lax.fori_loop(0, SUP, chunk_body, 0)
        return 0
    lax.fori_loop(0, NSUP, super_body, 0)

    plsc.subcore_barrier()
    pltpu.sync_copy(acc_o.at[pl.ds(sid * R_T, R_T)],
                    out_o.at[cid, pl.ds(sid * R_T, R_T)])
    pltpu.sync_copy(acc_i.at[pl.ds(sid * R_T, R_T)],
                    out_i.at[cid, pl.ds(sid * R_T, R_T)])


@functools.partial(
    pl.kernel,
    out_type=jax.ShapeDtypeStruct((NC, N_ACC, H), jnp.float32),
    mesh=_mesh,
    compiler_params=_sc_params,
    scratch_types=[
        pltpu.VMEM((SUP, CH), jnp.int32),
        pltpu.VMEM((SUP, CH), jnp.int32),
        pltpu.VMEM((CH, H), jnp.float32),
        pltpu.VMEM((ZR, H), jnp.float32),
        pltpu.VMEM_SHARED((N_ACC, H), jnp.float32),
        pltpu.SemaphoreType.DMA,
    ],
)
def _agg_call(src_hbm, dst_hbm, tab_hbm, out, sbuf, dbuf, rows_v, zb,
              acc, sem):
    cid = lax.axis_index("c")
    sid = lax.axis_index("s")

    def init_z(i, _):
        zb[i] = jnp.zeros((H,), jnp.float32)
        return 0
    lax.fori_loop(0, ZR, init_z, 0)

    def zero_acc(k, _):
        pltpu.sync_copy(zb, acc.at[pl.ds(sid * R_T + k * ZR, ZR)])
        return 0
    lax.fori_loop(0, R_T // ZR, zero_acc, 0)
    plsc.subcore_barrier()

    base_chunk = (cid * NS + sid) * CH_PER_TILE

    def super_body(s, _):
        pltpu.sync_copy(src_hbm.at[pl.ds(base_chunk + s * SUP, SUP)], sbuf)
        pltpu.sync_copy(dst_hbm.at[pl.ds(base_chunk + s * SUP, SUP)], dbuf)

        def chunk_body(j, _):
            pltpu.async_copy(tab_hbm.at[sbuf.at[j]], rows_v, sem).wait()
            pltpu.sync_copy(rows_v, acc.at[dbuf.at[j]], add=True)
            return 0
        lax.fori_loop(0, SUP, chunk_body, 0)
        return 0
    lax.fori_loop(0, NSUP, super_body, 0)

    plsc.subcore_barrier()
    pltpu.sync_copy(acc.at[pl.ds(sid * R_T, R_T)],
                    out.at[cid, pl.ds(sid * R_T, R_T)])


def _norm_body(dego, degi, e8, doutx, dinx):
    dn_o = lax.rsqrt(jnp.maximum(dego[0] + dego[1], 1.0))
    dn_i = lax.rsqrt(jnp.maximum(degi[0] + degi[1], 1.0))
    for s in range(16):
        doutx[:, s, :] = jnp.dot(dn_o[:, 8 * s:8 * s + 8], e8[...],
                                 preferred_element_type=jnp.float32)
        dinx[:, s, :] = jnp.dot(dn_i[:, 8 * s:8 * s + 8], e8[...],
                                preferred_element_type=jnp.float32)


def _norm_call(dego, degi, e8):
    return pl.pallas_call(
        _norm_body,
        grid=(1,),
        in_specs=[
            pl.BlockSpec((NC, GQ, 128), lambda i: (0, 0, 0)),
            pl.BlockSpec((NC, GQ, 128), lambda i: (0, 0, 0)),
            pl.BlockSpec((8, 128), lambda i: (0, 0)),
        ],
        out_specs=[
            pl.BlockSpec((GQ, 16, 128), lambda i: (0, 0, 0)),
            pl.BlockSpec((GQ, 16, 128), lambda i: (0, 0, 0)),
        ],
        out_shape=[
            jax.ShapeDtypeStruct((GQ, 16, 128), jnp.float32),
            jax.ShapeDtypeStruct((GQ, 16, 128), jnp.float32),
        ],
    )(dego, degi, e8)


def _prep_body(feat, doutx, bdw1, tab):
    tab[...] = jnp.dot(feat[...] * doutx[...], bdw1[...],
                       preferred_element_type=jnp.float32)


def _prep_call(featp, doutx, bdw1):
    return pl.pallas_call(
        _prep_body,
        grid=(NB,),
        in_specs=[
            pl.BlockSpec((BP, 128), lambda i: (i, 0)),
            pl.BlockSpec((BP, 128), lambda i: (i, 0)),
            pl.BlockSpec((128, 128), lambda i: (0, 0)),
        ],
        out_specs=pl.BlockSpec((BP, 128), lambda i: (i, 0)),
        out_shape=jax.ShapeDtypeStruct((NP8, 128), jnp.float32),
    )(featp, doutx, bdw1)


def _mid_body(agg, dinx, doutx, bdw2, b1t, tab2):
    h1 = jnp.maximum((agg[0] + agg[1]) * dinx[...] + b1t[...], 0.0)
    tab2[...] = jnp.dot(h1 * doutx[...], bdw2[...],
                        preferred_element_type=jnp.float32)


def _mid_call(agg1, dinx, doutx, bdw2, b1t):
    return pl.pallas_call(
        _mid_body,
        grid=(NB,),
        in_specs=[
            pl.BlockSpec((NC, BP, 128), lambda i: (0, i, 0)),
            pl.BlockSpec((BP, 128), lambda i: (i, 0)),
            pl.BlockSpec((BP, 128), lambda i: (i, 0)),
            pl.BlockSpec((128, 128), lambda i: (0, 0)),
            pl.BlockSpec((1, 128), lambda i: (0, 0)),
        ],
        out_specs=pl.BlockSpec((BP, 128), lambda i: (i, 0)),
        out_shape=jax.ShapeDtypeStruct((NP8, 128), jnp.float32),
    )(agg1, dinx, doutx, bdw2, b1t)


def _fin_body(agg, dinx, b2t, bdwr, bdwz, bdwn, brt, bzt, bn1t, bhnt,
              bdwfc, bfct, out):
    h2 = jnp.maximum((agg[0] + agg[1]) * dinx[...] + b2t[...], 0.0)
    r = jax.nn.sigmoid(jnp.dot(h2, bdwr[...],
                               preferred_element_type=jnp.float32) + brt[...])
    z = jax.nn.sigmoid(jnp.dot(h2, bdwz[...],
                               preferred_element_type=jnp.float32) + bzt[...])
    n = jnp.tanh(jnp.dot(h2, bdwn[...], preferred_element_type=jnp.float32)
                 + bn1t[...] + r * bhnt[...])
    hy = (1.0 - z) * n
    out[...] = jnp.dot(hy, bdwfc[...],
                       preferred_element_type=jnp.float32) + bfct[...]


def _fin_call(agg2, dinx, b2t, bdwr, bdwz, bdwn, brt, bzt, bn1t, bhnt,
              bdwfc, bfct):
    small = lambda r, c: pl.BlockSpec((r, c), lambda i: (0, 0))
    return pl.pallas_call(
        _fin_body,
        grid=(NB,),
        in_specs=[
            pl.BlockSpec((NC, BP, 128), lambda i: (0, i, 0)),
            pl.BlockSpec((BP, 128), lambda i: (i, 0)),
            small(1, 128), small(128, 128), small(128, 128), small(128, 128),
            small(1, 128), small(1, 128), small(1, 128), small(1, 128),
            small(128, 128), small(1, 128),
        ],
        out_specs=pl.BlockSpec((BP, 128), lambda i: (i, 0)),
        out_shape=jax.ShapeDtypeStruct((NP8, 128), jnp.float32),
    )(agg2, dinx, b2t, bdwr, bdwz, bdwn, brt, bzt, bn1t, bhnt, bdwfc, bfct)


def kernel(features, edge_index, W1, b1, W2, b2, W_ih, b_ih, W_hh, b_hh,
           W_fc, b_fc):
    src = edge_index[0]
    dst = edge_index[1]
    npad = E_PAD - E
    i = jnp.arange(npad, dtype=jnp.int32)
    pad_src = N + (i % (N_ACC - N))
    pad_dst = N + ((i * 7 + 3) % (N_ACC - N))
    src_p = jnp.concatenate([src, pad_src]).reshape(NCHUNK, CH)
    dst_p = jnp.concatenate([dst, pad_dst]).reshape(NCHUNK, CH)

    dego, degi = _deg_call(src_p, dst_p)

    eye8 = jnp.eye(8, dtype=jnp.float32)
    e8 = jnp.kron(eye8, jnp.ones((1, 16), jnp.float32))
    doutx3, dinx3 = _norm_call(dego.reshape(NC, GQ, 128),
                               degi.reshape(NC, GQ, 128), e8)
    doutx = doutx3.reshape(NP8, 128)
    dinx = dinx3.reshape(NP8, 128)

    featp = jnp.pad(features, ((0, N_ACC - N), (0, H - IN_F))) \
        .reshape(NP8, 128)
    w1p = jnp.zeros((H, H), jnp.float32).at[:IN_F].set(W1)
    bdw1 = jnp.kron(eye8, w1p)
    tab1 = _prep_call(featp, doutx, bdw1)

    agg1 = _agg_call(src_p, dst_p, tab1.reshape(N_ACC, H))
    bdw2 = jnp.kron(eye8, W2)
    tab2 = _mid_call(agg1.reshape(NC, NP8, 128), dinx, doutx, bdw2,
                     jnp.tile(b1, 8).reshape(1, 128))

    agg2 = _agg_call(src_p, dst_p, tab2.reshape(N_ACC, H))

    wfcp = jnp.zeros((H, H), jnp.float32).at[:, :1].set(W_fc.T)
    outp = _fin_call(
        agg2.reshape(NC, NP8, 128), dinx,
        jnp.tile(b2, 8).reshape(1, 128),
        jnp.kron(eye8, W_ih[:H].T), jnp.kron(eye8, W_ih[H:2 * H].T),
        jnp.kron(eye8, W_ih[2 * H:].T),
        jnp.tile(b_ih[:H] + b_hh[:H], 8).reshape(1, 128),
        jnp.tile(b_ih[H:2 * H] + b_hh[H:2 * H], 8).reshape(1, 128),
        jnp.tile(b_ih[2 * H:], 8).reshape(1, 128),
        jnp.tile(b_hh[2 * H:], 8).reshape(1, 128),
        jnp.kron(eye8, wfcp),
        jnp.tile(jnp.concatenate([b_fc, jnp.zeros((15,), jnp.float32)]),
                 8).reshape(1, 128),
    )
    return outp.reshape(N_ACC, H)[:N, :1]


# speedup vs baseline: 56.7382x; 2.2352x over previous
import functools

import jax
import jax.numpy as jnp
from jax import lax
from jax.experimental import pallas as pl
from jax.experimental.pallas import tpu as pltpu
from jax.experimental.pallas import tpu_sc as plsc

N = 100000
E = 3200000
IN_F = 3
H = 16

NC = 2
NS = 16
NW = NC * NS

N_ACC = 100352
R_T = N_ACC // NS
ZR = 392

CH = 128
SUP = 40
NSUP = 20
CH_PER_TILE = SUP * NSUP
NCHUNK = NW * CH_PER_TILE
E_PAD = NCHUNK * CH

NP8 = N_ACC // 8
GQ = N_ACC // 128
NB = 16
BP = NP8 // NB
BG = GQ // NB

_mesh = plsc.VectorSubcoreMesh(core_axis_name="c", subcore_axis_name="s",
                               num_cores=NC, num_subcores=NS)
_sc_params = pltpu.CompilerParams(use_tc_tiling_on_sc=False)


@functools.partial(
    pl.kernel,
    out_type=[jax.ShapeDtypeStruct((NC, N_ACC), jnp.float32),
              jax.ShapeDtypeStruct((NC, N_ACC), jnp.float32)],
    mesh=_mesh,
    compiler_params=_sc_params,
    scratch_types=[
        pltpu.VMEM((SUP, CH), jnp.int32),
        pltpu.VMEM((SUP, CH), jnp.int32),
        pltpu.VMEM((CH,), jnp.float32),
        pltpu.VMEM((R_T,), jnp.float32),
        pltpu.VMEM_SHARED((N_ACC,), jnp.float32),
        pltpu.VMEM_SHARED((N_ACC,), jnp.float32),
        pltpu.SemaphoreType.DMA,
    ],
)
def _deg_call(src_hbm, dst_hbm, out_o, out_i, sbuf, dbuf, ones_v, zb,
              acc_o, acc_i, dsem):
    cid = lax.axis_index("c")
    sid = lax.axis_index("s")

    def init_ones(i, _):
        ones_v[pl.ds(i * 16, 16)] = jnp.ones((16,), jnp.float32)
        return 0
    lax.fori_loop(0, CH // 16, init_ones, 0)

    def init_z(i, _):
        zb[pl.ds(i * 16, 16)] = jnp.zeros((16,), jnp.float32)
        return 0
    lax.fori_loop(0, R_T // 16, init_z, 0)

    pltpu.sync_copy(zb, acc_o.at[pl.ds(sid * R_T, R_T)])
    pltpu.sync_copy(zb, acc_i.at[pl.ds(sid * R_T, R_T)])
    plsc.subcore_barrier()

    base_chunk = (cid * NS + sid) * CH_PER_TILE

    def super_body(s, _):
        pltpu.sync_copy(src_hbm.at[pl.ds(base_chunk + s * SUP, SUP)], sbuf)
        pltpu.sync_copy(dst_hbm.at[pl.ds(base_chunk + s * SUP, SUP)], dbuf)

        def grp_body(t, _):
            descs = []
            for u in range(4):
                descs.append(pltpu.async_copy(
                    ones_v, acc_o.at[sbuf.at[4 * t + u]], dsem, add=True))
                descs.append(pltpu.async_copy(
                    ones_v, acc_i.at[dbuf.at[4 * t + u]], dsem, add=True))
            for d in descs:
                d.wait()
            return 0
        lax.fori_loop(0, SUP // 4, grp_body, 0)
        return 0
    lax.fori_loop(0, NSUP, super_body, 0)

    plsc.subcore_barrier()
    pltpu.sync_copy(acc_o.at[pl.ds(sid * R_T, R_T)],
                    out_o.at[cid, pl.ds(sid * R_T, R_T)])
    pltpu.sync_copy(acc_i.at[pl.ds(sid * R_T, R_T)],
                    out_i.at[cid, pl.ds(sid * R_T, R_T)])


@functools.partial(
    pl.kernel,
    out_type=jax.ShapeDtypeStruct((NC, N_ACC, H), jnp.float32),
    mesh=_mesh,
    compiler_params=_sc_params,
    scratch_types=[
        pltpu.VMEM((SUP, CH), jnp.int32),
        pltpu.VMEM((SUP, CH), jnp.int32),
        pltpu.VMEM((4 * CH, H), jnp.float32),
        pltpu.VMEM((4 * CH, H), jnp.float32),
        pltpu.VMEM_SHARED((N_ACC, H), jnp.float32),
        pltpu.SemaphoreType.DMA,
        pltpu.SemaphoreType.DMA,
    ],
)
def _agg_call(src_hbm, dst_hbm, tab_hbm, out, sbuf, dbuf, rows_a, rows_b,
              acc, gsa, gsb):
    cid = lax.axis_index("c")
    sid = lax.axis_index("s")

    def init_z(i, _):
        rows_a[i] = jnp.zeros((H,), jnp.float32)
        return 0
    lax.fori_loop(0, 4 * CH, init_z, 0)

    def zero_acc(k, _):
        pltpu.sync_copy(rows_a, acc.at[pl.ds(sid * R_T + k * 4 * CH, 4 * CH)])
        return 0
    lax.fori_loop(0, R_T // (4 * CH), zero_acc, 0)
    pltpu.sync_copy(rows_a.at[pl.ds(0, CH)],
                    acc.at[pl.ds(sid * R_T + (R_T // (4 * CH)) * 4 * CH, CH)])
    plsc.subcore_barrier()

    base_chunk = (cid * NS + sid) * CH_PER_TILE

    def super_body(s, _):
        pltpu.sync_copy(src_hbm.at[pl.ds(base_chunk + s * SUP, SUP)], sbuf)
        pltpu.sync_copy(dst_hbm.at[pl.ds(base_chunk + s * SUP, SUP)], dbuf)

        for u in range(4):
            pltpu.async_copy(tab_hbm.at[sbuf.at[u]],
                             rows_a.at[pl.ds(CH * u, CH)], gsa)

        def grp_pair(t, _):
            b0 = 8 * t
            for u in range(4):
                pltpu.async_copy(tab_hbm.at[sbuf.at[b0 + 4 + u]],
                                 rows_b.at[pl.ds(CH * u, CH)], gsb)
            for u in range(4):
                pltpu.make_async_copy(tab_hbm.at[sbuf.at[b0 + u]],
                                      rows_a.at[pl.ds(CH * u, CH)],
                                      gsa).wait()
            for u in range(4):
                pltpu.sync_copy(rows_a.at[pl.ds(CH * u, CH)],
                                acc.at[dbuf.at[b0 + u]], add=True)

            @pl.when(t < SUP // 8 - 1)
            def _():
                for u in range(4):
                    pltpu.async_copy(tab_hbm.at[sbuf.at[b0 + 8 + u]],
                                     rows_a.at[pl.ds(CH * u, CH)], gsa)
            for u in range(4):
                pltpu.make_async_copy(tab_hbm.at[sbuf.at[b0 + 4 + u]],
                                      rows_b.at[pl.ds(CH * u, CH)],
                                      gsb).wait()
            for u in range(4):
                pltpu.sync_copy(rows_b.at[pl.ds(CH * u, CH)],
                                acc.at[dbuf.at[b0 + 4 + u]], add=True)
            return 0
        lax.fori_loop(0, SUP // 8, grp_pair, 0)
        return 0
    lax.fori_loop(0, NSUP, super_body, 0)

    plsc.subcore_barrier()
    pltpu.sync_copy(acc.at[pl.ds(sid * R_T, R_T)],
                    out.at[cid, pl.ds(sid * R_T, R_T)])


def _norm_body(dego, degi, e8, doutx, dinx):
    dn_o = lax.rsqrt(jnp.maximum(dego[0] + dego[1], 1.0))
    dn_i = lax.rsqrt(jnp.maximum(degi[0] + degi[1], 1.0))
    for s in range(16):
        doutx[:, s, :] = jnp.dot(dn_o[:, 8 * s:8 * s + 8], e8[...],
                                 preferred_element_type=jnp.float32,
                       precision=lax.Precision.HIGHEST)
        dinx[:, s, :] = jnp.dot(dn_i[:, 8 * s:8 * s + 8], e8[...],
                                preferred_element_type=jnp.float32,
                       precision=lax.Precision.HIGHEST)


def _norm_call(dego, degi, e8):
    return pl.pallas_call(
        _norm_body,
        grid=(1,),
        in_specs=[
            pl.BlockSpec((NC, GQ, 128), lambda i: (0, 0, 0)),
            pl.BlockSpec((NC, GQ, 128), lambda i: (0, 0, 0)),
            pl.BlockSpec((8, 128), lambda i: (0, 0)),
        ],
        out_specs=[
            pl.BlockSpec((GQ, 16, 128), lambda i: (0, 0, 0)),
            pl.BlockSpec((GQ, 16, 128), lambda i: (0, 0, 0)),
        ],
        out_shape=[
            jax.ShapeDtypeStruct((GQ, 16, 128), jnp.float32),
            jax.ShapeDtypeStruct((GQ, 16, 128), jnp.float32),
        ],
    )(dego, degi, e8)


def _prep_body(feat, doutx, tab):
    tab[...] = feat[...] * doutx[...]


def _prep_call(featp, doutx):
    return pl.pallas_call(
        _prep_body,
        grid=(NB,),
        in_specs=[
            pl.BlockSpec((BP, 128), lambda i: (i, 0)),
            pl.BlockSpec((BP, 128), lambda i: (i, 0)),
        ],
        out_specs=pl.BlockSpec((BP, 128), lambda i: (i, 0)),
        out_shape=jax.ShapeDtypeStruct((NP8, 128), jnp.float32),
    )(featp, doutx)


def _bdot(x, w_bf):
    return jnp.dot(x.astype(jnp.bfloat16), w_bf,
                   preferred_element_type=jnp.float32)


def _mid_body(agg, dinx, doutx, bdw1, b1t, tab2):
    t = (agg[0] + agg[1]) * dinx[...]
    h1 = jnp.maximum(_bdot(t, bdw1[...]) + b1t[...], 0.0)
    tab2[...] = h1 * doutx[...]


def _mid_call(agg1, dinx, doutx, bdw1, b1t):
    return pl.pallas_call(
        _mid_body,
        grid=(NB,),
        in_specs=[
            pl.BlockSpec((NC, BP, 128), lambda i: (0, i, 0)),
            pl.BlockSpec((BP, 128), lambda i: (i, 0)),
            pl.BlockSpec((BP, 128), lambda i: (i, 0)),
            pl.BlockSpec((128, 128), lambda i: (0, 0)),
            pl.BlockSpec((1, 128), lambda i: (0, 0)),
        ],
        out_specs=pl.BlockSpec((BP, 128), lambda i: (i, 0)),
        out_shape=jax.ShapeDtypeStruct((NP8, 128), jnp.float32),
    )(agg1, dinx, doutx, bdw1, b1t)


def _fin_body(agg, dinx, bdw2, b2t, bdwr, bdwz, bdwn, brt, bzt, bn1t, bhnt,
              bdwfc, bfct, out):
    t = (agg[0] + agg[1]) * dinx[...]
    h2 = jnp.maximum(_bdot(t, bdw2[...]) + b2t[...], 0.0)
    h2b = h2.astype(jnp.bfloat16)
    r = jax.nn.sigmoid(jnp.dot(h2b, bdwr[...],
                               preferred_element_type=jnp.float32) + brt[...])
    z = jax.nn.sigmoid(jnp.dot(h2b, bdwz[...],
                               preferred_element_type=jnp.float32) + bzt[...])
    n = jnp.tanh(jnp.dot(h2b, bdwn[...], preferred_element_type=jnp.float32)
                 + bn1t[...] + r * bhnt[...])
    hy = (1.0 - z) * n
    out[...] = _bdot(hy, bdwfc[...]) + bfct[...]


def _fin_call(agg2, dinx, bdw2, b2t, bdwr, bdwz, bdwn, brt, bzt, bn1t, bhnt,
              bdwfc, bfct):
    small = lambda r, c: pl.BlockSpec((r, c), lambda i: (0, 0))
    return pl.pallas_call(
        _fin_body,
        grid=(NB,),
        in_specs=[
            pl.BlockSpec((NC, BP, 128), lambda i: (0, i, 0)),
            pl.BlockSpec((BP, 128), lambda i: (i, 0)),
            small(128, 128), small(1, 128),
            small(128, 128), small(128, 128), small(128, 128),
            small(1, 128), small(1, 128), small(1, 128), small(1, 128),
            small(128, 128), small(1, 128),
        ],
        out_specs=pl.BlockSpec((BP, 128), lambda i: (i, 0)),
        out_shape=jax.ShapeDtypeStruct((NP8, 128), jnp.float32),
    )(agg2, dinx, bdw2, b2t, bdwr, bdwz, bdwn, brt, bzt, bn1t, bhnt,
      bdwfc, bfct)


def kernel(features, edge_index, W1, b1, W2, b2, W_ih, b_ih, W_hh, b_hh,
           W_fc, b_fc):
    src = edge_index[0]
    dst = edge_index[1]
    npad = E_PAD - E
    i = jnp.arange(npad, dtype=jnp.int32)
    pad_src = N + (i % (N_ACC - N))
    pad_dst = N + ((i * 7 + 3) % (N_ACC - N))
    src_p = jnp.concatenate([src, pad_src]).reshape(NCHUNK, CH)
    dst_p = jnp.concatenate([dst, pad_dst]).reshape(NCHUNK, CH)

    dego, degi = _deg_call(src_p, dst_p)

    eye8 = jnp.eye(8, dtype=jnp.float32)
    e8 = jnp.kron(eye8, jnp.ones((1, 16), jnp.float32))
    doutx3, dinx3 = _norm_call(dego.reshape(NC, GQ, 128),
                               degi.reshape(NC, GQ, 128), e8)
    doutx = doutx3.reshape(NP8, 128)
    dinx = dinx3.reshape(NP8, 128)

    featp = jnp.pad(features, ((0, N_ACC - N), (0, H - IN_F))) \
        .reshape(NP8, 128)
    tab1 = _prep_call(featp, doutx)

    agg1 = _agg_call(src_p, dst_p, tab1.reshape(N_ACC, H))

    bf = jnp.bfloat16
    w1p = jnp.zeros((H, H), jnp.float32).at[:IN_F].set(W1)
    tab2 = _mid_call(agg1.reshape(NC, NP8, 128), dinx, doutx,
                     jnp.kron(eye8, w1p).astype(bf),
                     jnp.tile(b1, 8).reshape(1, 128))

    agg2 = _agg_call(src_p, dst_p, tab2.reshape(N_ACC, H))

    wfcp = jnp.zeros((H, H), jnp.float32).at[:, :1].set(W_fc.T)
    outp = _fin_call(
        agg2.reshape(NC, NP8, 128), dinx,
        jnp.kron(eye8, W2).astype(bf),
        jnp.tile(b2, 8).reshape(1, 128),
        jnp.kron(eye8, W_ih[:H].T).astype(bf),
        jnp.kron(eye8, W_ih[H:2 * H].T).astype(bf),
        jnp.kron(eye8, W_ih[2 * H:].T).astype(bf),
        jnp.tile(b_ih[:H] + b_hh[:H], 8).reshape(1, 128),
        jnp.tile(b_ih[H:2 * H] + b_hh[H:2 * H], 8).reshape(1, 128),
        jnp.tile(b_ih[2 * H:], 8).reshape(1, 128),
        jnp.tile(b_hh[2 * H:], 8).reshape(1, 128),
        jnp.kron(eye8, wfcp).astype(bf),
        jnp.tile(jnp.concatenate([b_fc, jnp.zeros((15,), jnp.float32)]),
                 8).reshape(1, 128),
    )
    return outp.reshape(N_ACC, H)[:N, :1]


# speedup vs baseline: 57.9352x; 1.0211x over previous
import functools

import jax
import jax.numpy as jnp
from jax import lax
from jax.experimental import pallas as pl
from jax.experimental.pallas import tpu as pltpu
from jax.experimental.pallas import tpu_sc as plsc

N = 100000
E = 3200000
IN_F = 3
H = 16

NC = 2
NS = 16
NW = NC * NS

N_ACC = 100352
R_T = N_ACC // NS
ZR = 392

CH = 128
SUP = 40
NSUP = 20
CH_PER_TILE = SUP * NSUP
NCHUNK = NW * CH_PER_TILE
E_PAD = NCHUNK * CH

NP8 = N_ACC // 8
GQ = N_ACC // 128
NB = 16
BP = NP8 // NB
BG = GQ // NB

_mesh = plsc.VectorSubcoreMesh(core_axis_name="c", subcore_axis_name="s",
                               num_cores=NC, num_subcores=NS)
_sc_params = pltpu.CompilerParams(use_tc_tiling_on_sc=False)


@functools.partial(
    pl.kernel,
    out_type=[jax.ShapeDtypeStruct((NC, N_ACC), jnp.float32),
              jax.ShapeDtypeStruct((NC, N_ACC), jnp.float32)],
    mesh=_mesh,
    compiler_params=_sc_params,
    scratch_types=[
        pltpu.VMEM((SUP, CH), jnp.int32),
        pltpu.VMEM((SUP, CH), jnp.int32),
        pltpu.VMEM((CH,), jnp.float32),
        pltpu.VMEM((R_T,), jnp.float32),
        pltpu.VMEM_SHARED((N_ACC,), jnp.float32),
        pltpu.VMEM_SHARED((N_ACC,), jnp.float32),
        pltpu.SemaphoreType.DMA,
    ],
)
def _deg_call(src_hbm, dst_hbm, out_o, out_i, sbuf, dbuf, ones_v, zb,
              acc_o, acc_i, dsem):
    cid = lax.axis_index("c")
    sid = lax.axis_index("s")

    def init_ones(i, _):
        ones_v[pl.ds(i * 16, 16)] = jnp.ones((16,), jnp.float32)
        return 0
    lax.fori_loop(0, CH // 16, init_ones, 0)

    def init_z(i, _):
        zb[pl.ds(i * 16, 16)] = jnp.zeros((16,), jnp.float32)
        return 0
    lax.fori_loop(0, R_T // 16, init_z, 0)

    pltpu.sync_copy(zb, acc_o.at[pl.ds(sid * R_T, R_T)])
    pltpu.sync_copy(zb, acc_i.at[pl.ds(sid * R_T, R_T)])
    plsc.subcore_barrier()

    base_chunk = (cid * NS + sid) * CH_PER_TILE

    def super_body(s, _):
        pltpu.sync_copy(src_hbm.at[pl.ds(base_chunk + s * SUP, SUP)], sbuf)
        pltpu.sync_copy(dst_hbm.at[pl.ds(base_chunk + s * SUP, SUP)], dbuf)

        def grp_body(t, _):
            descs = []
            for u in range(4):
                descs.append(pltpu.async_copy(
                    ones_v, acc_o.at[sbuf.at[4 * t + u]], dsem, add=True))
                descs.append(pltpu.async_copy(
                    ones_v, acc_i.at[dbuf.at[4 * t + u]], dsem, add=True))
            for d in descs:
                d.wait()
            return 0
        lax.fori_loop(0, SUP // 4, grp_body, 0)
        return 0
    lax.fori_loop(0, NSUP, super_body, 0)

    plsc.subcore_barrier()
    pltpu.sync_copy(acc_o.at[pl.ds(sid * R_T, R_T)],
                    out_o.at[cid, pl.ds(sid * R_T, R_T)])
    pltpu.sync_copy(acc_i.at[pl.ds(sid * R_T, R_T)],
                    out_i.at[cid, pl.ds(sid * R_T, R_T)])


@functools.partial(
    pl.kernel,
    out_type=jax.ShapeDtypeStruct((NC, N_ACC, H), jnp.float32),
    mesh=_mesh,
    compiler_params=_sc_params,
    scratch_types=[
        pltpu.VMEM((SUP, CH), jnp.int32),
        pltpu.VMEM((SUP, CH), jnp.int32),
        pltpu.VMEM((5 * CH, H), jnp.float32),
        pltpu.VMEM((5 * CH, H), jnp.float32),
        pltpu.VMEM_SHARED((N_ACC, H), jnp.float32),
        pltpu.SemaphoreType.DMA,
        pltpu.SemaphoreType.DMA,
    ],
)
def _agg_call(src_hbm, dst_hbm, tab_hbm, out, sbuf, dbuf, rows_a, rows_b,
              acc, gsa, gsb):
    cid = lax.axis_index("c")
    sid = lax.axis_index("s")

    def init_z(i, _):
        rows_a[i] = jnp.zeros((H,), jnp.float32)
        return 0
    lax.fori_loop(0, 5 * CH, init_z, 0)

    def zero_acc(k, _):
        pltpu.sync_copy(rows_a, acc.at[pl.ds(sid * R_T + k * 5 * CH, 5 * CH)])
        return 0
    lax.fori_loop(0, R_T // (5 * CH), zero_acc, 0)
    pltpu.sync_copy(rows_a.at[pl.ds(0, 4 * CH)],
                    acc.at[pl.ds(sid * R_T + (R_T // (5 * CH)) * 5 * CH,
                                 4 * CH)])
    plsc.subcore_barrier()

    base_chunk = (cid * NS + sid) * CH_PER_TILE

    def super_body(s, _):
        pltpu.sync_copy(src_hbm.at[pl.ds(base_chunk + s * SUP, SUP)], sbuf)
        pltpu.sync_copy(dst_hbm.at[pl.ds(base_chunk + s * SUP, SUP)], dbuf)

        G = 5
        for u in range(G):
            pltpu.async_copy(tab_hbm.at[sbuf.at[u]],
                             rows_a.at[pl.ds(CH * u, CH)], gsa)

        def grp_pair(t, _):
            b0 = 2 * G * t
            for u in range(G):
                pltpu.async_copy(tab_hbm.at[sbuf.at[b0 + G + u]],
                                 rows_b.at[pl.ds(CH * u, CH)], gsb)
            for u in range(G):
                pltpu.make_async_copy(tab_hbm.at[sbuf.at[b0 + u]],
                                      rows_a.at[pl.ds(CH * u, CH)],
                                      gsa).wait()
            for u in range(G):
                pltpu.sync_copy(rows_a.at[pl.ds(CH * u, CH)],
                                acc.at[dbuf.at[b0 + u]], add=True)

            @pl.when(t < SUP // (2 * G) - 1)
            def _():
                for u in range(G):
                    pltpu.async_copy(tab_hbm.at[sbuf.at[b0 + 2 * G + u]],
                                     rows_a.at[pl.ds(CH * u, CH)], gsa)
            for u in range(G):
                pltpu.make_async_copy(tab_hbm.at[sbuf.at[b0 + G + u]],
                                      rows_b.at[pl.ds(CH * u, CH)],
                                      gsb).wait()
            for u in range(G):
                pltpu.sync_copy(rows_b.at[pl.ds(CH * u, CH)],
                                acc.at[dbuf.at[b0 + G + u]], add=True)
            return 0
        lax.fori_loop(0, SUP // (2 * G), grp_pair, 0)
        return 0
    lax.fori_loop(0, NSUP, super_body, 0)

    plsc.subcore_barrier()
    pltpu.sync_copy(acc.at[pl.ds(sid * R_T, R_T)],
                    out.at[cid, pl.ds(sid * R_T, R_T)])


def _norm_body(dego, degi, e8, doutx, dinx):
    dn_o = lax.rsqrt(jnp.maximum(dego[0] + dego[1], 1.0))
    dn_i = lax.rsqrt(jnp.maximum(degi[0] + degi[1], 1.0))
    for s in range(16):
        doutx[:, s, :] = jnp.dot(dn_o[:, 8 * s:8 * s + 8], e8[...],
                                 preferred_element_type=jnp.float32,
                       precision=lax.Precision.HIGHEST)
        dinx[:, s, :] = jnp.dot(dn_i[:, 8 * s:8 * s + 8], e8[...],
                                preferred_element_type=jnp.float32,
                       precision=lax.Precision.HIGHEST)


def _norm_call(dego, degi, e8):
    return pl.pallas_call(
        _norm_body,
        grid=(1,),
        in_specs=[
            pl.BlockSpec((NC, GQ, 128), lambda i: (0, 0, 0)),
            pl.BlockSpec((NC, GQ, 128), lambda i: (0, 0, 0)),
            pl.BlockSpec((8, 128), lambda i: (0, 0)),
        ],
        out_specs=[
            pl.BlockSpec((GQ, 16, 128), lambda i: (0, 0, 0)),
            pl.BlockSpec((GQ, 16, 128), lambda i: (0, 0, 0)),
        ],
        out_shape=[
            jax.ShapeDtypeStruct((GQ, 16, 128), jnp.float32),
            jax.ShapeDtypeStruct((GQ, 16, 128), jnp.float32),
        ],
    )(dego, degi, e8)


def _prep_body(feat, doutx, tab):
    tab[...] = feat[...] * doutx[...]


def _prep_call(featp, doutx):
    return pl.pallas_call(
        _prep_body,
        grid=(NB,),
        in_specs=[
            pl.BlockSpec((BP, 128), lambda i: (i, 0)),
            pl.BlockSpec((BP, 128), lambda i: (i, 0)),
        ],
        out_specs=pl.BlockSpec((BP, 128), lambda i: (i, 0)),
        out_shape=jax.ShapeDtypeStruct((NP8, 128), jnp.float32),
    )(featp, doutx)


def _bdot(x, w_bf):
    return jnp.dot(x.astype(jnp.bfloat16), w_bf,
                   preferred_element_type=jnp.float32)


def _mid_body(agg, dinx, doutx, bdw1, b1t, tab2):
    t = (agg[0] + agg[1]) * dinx[...]
    h1 = jnp.maximum(_bdot(t, bdw1[...]) + b1t[...], 0.0)
    tab2[...] = h1 * doutx[...]


def _mid_call(agg1, dinx, doutx, bdw1, b1t):
    return pl.pallas_call(
        _mid_body,
        grid=(NB,),
        in_specs=[
            pl.BlockSpec((NC, BP, 128), lambda i: (0, i, 0)),
            pl.BlockSpec((BP, 128), lambda i: (i, 0)),
            pl.BlockSpec((BP, 128), lambda i: (i, 0)),
            pl.BlockSpec((128, 128), lambda i: (0, 0)),
            pl.BlockSpec((1, 128), lambda i: (0, 0)),
        ],
        out_specs=pl.BlockSpec((BP, 128), lambda i: (i, 0)),
        out_shape=jax.ShapeDtypeStruct((NP8, 128), jnp.float32),
    )(agg1, dinx, doutx, bdw1, b1t)


def _fin_body(agg, dinx, bdw2, b2t, bdwr, bdwz, bdwn, brt, bzt, bn1t, bhnt,
              bdwfc, bfct, out):
    t = (agg[0] + agg[1]) * dinx[...]
    h2 = jnp.maximum(_bdot(t, bdw2[...]) + b2t[...], 0.0)
    h2b = h2.astype(jnp.bfloat16)
    r = jax.nn.sigmoid(jnp.dot(h2b, bdwr[...],
                               preferred_element_type=jnp.float32) + brt[...])
    z = jax.nn.sigmoid(jnp.dot(h2b, bdwz[...],
                               preferred_element_type=jnp.float32) + bzt[...])
    n = jnp.tanh(jnp.dot(h2b, bdwn[...], preferred_element_type=jnp.float32)
                 + bn1t[...] + r * bhnt[...])
    hy = (1.0 - z) * n
    out[...] = _bdot(hy, bdwfc[...]) + bfct[...]


def _fin_call(agg2, dinx, bdw2, b2t, bdwr, bdwz, bdwn, brt, bzt, bn1t, bhnt,
              bdwfc, bfct):
    small = lambda r, c: pl.BlockSpec((r, c), lambda i: (0, 0))
    return pl.pallas_call(
        _fin_body,
        grid=(NB,),
        in_specs=[
            pl.BlockSpec((NC, BP, 128), lambda i: (0, i, 0)),
            pl.BlockSpec((BP, 128), lambda i: (i, 0)),
            small(128, 128), small(1, 128),
            small(128, 128), small(128, 128), small(128, 128),
            small(1, 128), small(1, 128), small(1, 128), small(1, 128),
            small(128, 128), small(1, 128),
        ],
        out_specs=pl.BlockSpec((BP, 128), lambda i: (i, 0)),
        out_shape=jax.ShapeDtypeStruct((NP8, 128), jnp.float32),
    )(agg2, dinx, bdw2, b2t, bdwr, bdwz, bdwn, brt, bzt, bn1t, bhnt,
      bdwfc, bfct)


def kernel(features, edge_index, W1, b1, W2, b2, W_ih, b_ih, W_hh, b_hh,
           W_fc, b_fc):
    src = edge_index[0]
    dst = edge_index[1]
    npad = E_PAD - E
    i = jnp.arange(npad, dtype=jnp.int32)
    pad_src = N + (i % (N_ACC - N))
    pad_dst = N + ((i * 7 + 3) % (N_ACC - N))
    src_p = jnp.concatenate([src, pad_src]).reshape(NCHUNK, CH)
    dst_p = jnp.concatenate([dst, pad_dst]).reshape(NCHUNK, CH)

    dego, degi = _deg_call(src_p, dst_p)

    eye8 = jnp.eye(8, dtype=jnp.float32)
    e8 = jnp.kron(eye8, jnp.ones((1, 16), jnp.float32))
    doutx3, dinx3 = _norm_call(dego.reshape(NC, GQ, 128),
                               degi.reshape(NC, GQ, 128), e8)
    doutx = doutx3.reshape(NP8, 128)
    dinx = dinx3.reshape(NP8, 128)

    featp = jnp.pad(features, ((0, N_ACC - N), (0, H - IN_F))) \
        .reshape(NP8, 128)
    tab1 = _prep_call(featp, doutx)

    agg1 = _agg_call(src_p, dst_p, tab1.reshape(N_ACC, H))

    bf = jnp.bfloat16
    w1p = jnp.zeros((H, H), jnp.float32).at[:IN_F].set(W1)
    tab2 = _mid_call(agg1.reshape(NC, NP8, 128), dinx, doutx,
                     jnp.kron(eye8, w1p).astype(bf),
                     jnp.tile(b1, 8).reshape(1, 128))

    agg2 = _agg_call(src_p, dst_p, tab2.reshape(N_ACC, H))

    wfcp = jnp.zeros((H, H), jnp.float32).at[:, :1].set(W_fc.T)
    outp = _fin_call(
        agg2.reshape(NC, NP8, 128), dinx,
        jnp.kron(eye8, W2).astype(bf),
        jnp.tile(b2, 8).reshape(1, 128),
        jnp.kron(eye8, W_ih[:H].T).astype(bf),
        jnp.kron(eye8, W_ih[H:2 * H].T).astype(bf),
        jnp.kron(eye8, W_ih[2 * H:].T).astype(bf),
        jnp.tile(b_ih[:H] + b_hh[:H], 8).reshape(1, 128),
        jnp.tile(b_ih[H:2 * H] + b_hh[H:2 * H], 8).reshape(1, 128),
        jnp.tile(b_ih[2 * H:], 8).reshape(1, 128),
        jnp.tile(b_hh[2 * H:], 8).reshape(1, 128),
        jnp.kron(eye8, wfcp).astype(bf),
        jnp.tile(jnp.concatenate([b_fc, jnp.zeros((15,), jnp.float32)]),
                 8).reshape(1, 128),
    )
    return outp.reshape(N_ACC, H)[:N, :1]


# speedup vs baseline: 58.2585x; 1.0056x over previous
import functools

import jax
import jax.numpy as jnp
from jax import lax
from jax.experimental import pallas as pl
from jax.experimental.pallas import tpu as pltpu
from jax.experimental.pallas import tpu_sc as plsc

N = 100000
E = 3200000
IN_F = 3
H = 16

NC = 2
NS = 16
NW = NC * NS

N_ACC = 100352
R_T = N_ACC // NS
ZR = 392

CH = 128
SUP = 40
NSUP = 20
CH_PER_TILE = SUP * NSUP
NCHUNK = NW * CH_PER_TILE
E_PAD = NCHUNK * CH

NP8 = N_ACC // 8
GQ = N_ACC // 128
NB = 16
BP = NP8 // NB
BG = GQ // NB

_mesh = plsc.VectorSubcoreMesh(core_axis_name="c", subcore_axis_name="s",
                               num_cores=NC, num_subcores=NS)
_sc_params = pltpu.CompilerParams(use_tc_tiling_on_sc=False)


@functools.partial(
    pl.kernel,
    out_type=[jax.ShapeDtypeStruct((NC, N_ACC), jnp.float32),
              jax.ShapeDtypeStruct((NC, N_ACC), jnp.float32)],
    mesh=_mesh,
    compiler_params=_sc_params,
    scratch_types=[
        pltpu.VMEM((SUP, CH), jnp.int32),
        pltpu.VMEM((SUP, CH), jnp.int32),
        pltpu.VMEM((CH,), jnp.float32),
        pltpu.VMEM((R_T,), jnp.float32),
        pltpu.VMEM_SHARED((N_ACC,), jnp.float32),
        pltpu.VMEM_SHARED((N_ACC,), jnp.float32),
        pltpu.SemaphoreType.DMA,
    ],
)
def _deg_call(src_hbm, dst_hbm, out_o, out_i, sbuf, dbuf, ones_v, zb,
              acc_o, acc_i, dsem):
    cid = lax.axis_index("c")
    sid = lax.axis_index("s")

    def init_ones(i, _):
        ones_v[pl.ds(i * 16, 16)] = jnp.ones((16,), jnp.float32)
        return 0
    lax.fori_loop(0, CH // 16, init_ones, 0)

    def init_z(i, _):
        zb[pl.ds(i * 16, 16)] = jnp.zeros((16,), jnp.float32)
        return 0
    lax.fori_loop(0, R_T // 16, init_z, 0)

    pltpu.sync_copy(zb, acc_o.at[pl.ds(sid * R_T, R_T)])
    pltpu.sync_copy(zb, acc_i.at[pl.ds(sid * R_T, R_T)])
    plsc.subcore_barrier()

    base_chunk = (cid * NS + sid) * CH_PER_TILE

    def super_body(s, _):
        pltpu.sync_copy(src_hbm.at[pl.ds(base_chunk + s * SUP, SUP)], sbuf)
        pltpu.sync_copy(dst_hbm.at[pl.ds(base_chunk + s * SUP, SUP)], dbuf)

        def grp_body(t, _):
            descs = []
            for u in range(8):
                descs.append(pltpu.async_copy(
                    ones_v, acc_o.at[sbuf.at[8 * t + u]], dsem, add=True))
                descs.append(pltpu.async_copy(
                    ones_v, acc_i.at[dbuf.at[8 * t + u]], dsem, add=True))
            for d in descs:
                d.wait()
            return 0
        lax.fori_loop(0, SUP // 8, grp_body, 0)
        return 0
    lax.fori_loop(0, NSUP, super_body, 0)

    plsc.subcore_barrier()
    pltpu.sync_copy(acc_o.at[pl.ds(sid * R_T, R_T)],
                    out_o.at[cid, pl.ds(sid * R_T, R_T)])
    pltpu.sync_copy(acc_i.at[pl.ds(sid * R_T, R_T)],
                    out_i.at[cid, pl.ds(sid * R_T, R_T)])


@functools.partial(
    pl.kernel,
    out_type=jax.ShapeDtypeStruct((NC, N_ACC, H), jnp.float32),
    mesh=_mesh,
    compiler_params=_sc_params,
    scratch_types=[
        pltpu.VMEM((SUP, CH), jnp.int32),
        pltpu.VMEM((SUP, CH), jnp.int32),
        pltpu.VMEM((5 * CH, H), jnp.float32),
        pltpu.VMEM((5 * CH, H), jnp.float32),
        pltpu.VMEM_SHARED((N_ACC, H), jnp.float32),
        pltpu.SemaphoreType.DMA,
        pltpu.SemaphoreType.DMA,
    ],
)
def _agg_call(src_hbm, dst_hbm, tab_hbm, out, sbuf, dbuf, rows_a, rows_b,
              acc, gsa, gsb):
    cid = lax.axis_index("c")
    sid = lax.axis_index("s")

    def init_z(i, _):
        rows_a[i] = jnp.zeros((H,), jnp.float32)
        return 0
    lax.fori_loop(0, 5 * CH, init_z, 0)

    def zero_acc(k, _):
        pltpu.sync_copy(rows_a, acc.at[pl.ds(sid * R_T + k * 5 * CH, 5 * CH)])
        return 0
    lax.fori_loop(0, R_T // (5 * CH), zero_acc, 0)
    pltpu.sync_copy(rows_a.at[pl.ds(0, 4 * CH)],
                    acc.at[pl.ds(sid * R_T + (R_T // (5 * CH)) * 5 * CH,
                                 4 * CH)])
    plsc.subcore_barrier()

    base_chunk = (cid * NS + sid) * CH_PER_TILE

    def super_body(s, _):
        pltpu.sync_copy(src_hbm.at[pl.ds(base_chunk + s * SUP, SUP)], sbuf)
        pltpu.sync_copy(dst_hbm.at[pl.ds(base_chunk + s * SUP, SUP)], dbuf)

        G = 5
        for u in range(G):
            pltpu.async_copy(tab_hbm.at[sbuf.at[u]],
                             rows_a.at[pl.ds(CH * u, CH)], gsa)

        def grp_pair(t, _):
            b0 = 2 * G * t
            for u in range(G):
                pltpu.async_copy(tab_hbm.at[sbuf.at[b0 + G + u]],
                                 rows_b.at[pl.ds(CH * u, CH)], gsb)
            for u in range(G):
                pltpu.make_async_copy(tab_hbm.at[sbuf.at[b0 + u]],
                                      rows_a.at[pl.ds(CH * u, CH)],
                                      gsa).wait()
            for u in range(G):
                pltpu.sync_copy(rows_a.at[pl.ds(CH * u, CH)],
                                acc.at[dbuf.at[b0 + u]], add=True)

            @pl.when(t < SUP // (2 * G) - 1)
            def _():
                for u in range(G):
                    pltpu.async_copy(tab_hbm.at[sbuf.at[b0 + 2 * G + u]],
                                     rows_a.at[pl.ds(CH * u, CH)], gsa)
            for u in range(G):
                pltpu.make_async_copy(tab_hbm.at[sbuf.at[b0 + G + u]],
                                      rows_b.at[pl.ds(CH * u, CH)],
                                      gsb).wait()
            for u in range(G):
                pltpu.sync_copy(rows_b.at[pl.ds(CH * u, CH)],
                                acc.at[dbuf.at[b0 + G + u]], add=True)
            return 0
        lax.fori_loop(0, SUP // (2 * G), grp_pair, 0)
        return 0
    lax.fori_loop(0, NSUP, super_body, 0)

    plsc.subcore_barrier()
    pltpu.sync_copy(acc.at[pl.ds(sid * R_T, R_T)],
                    out.at[cid, pl.ds(sid * R_T, R_T)])


def _norm_body(dego, degi, e8, doutx, dinx):
    dn_o = lax.rsqrt(jnp.maximum(dego[0] + dego[1], 1.0))
    dn_i = lax.rsqrt(jnp.maximum(degi[0] + degi[1], 1.0))
    for s in range(16):
        doutx[:, s, :] = jnp.dot(dn_o[:, 8 * s:8 * s + 8], e8[...],
                                 preferred_element_type=jnp.float32,
                       precision=lax.Precision.HIGHEST)
        dinx[:, s, :] = jnp.dot(dn_i[:, 8 * s:8 * s + 8], e8[...],
                                preferred_element_type=jnp.float32,
                       precision=lax.Precision.HIGHEST)


def _norm_call(dego, degi, e8):
    return pl.pallas_call(
        _norm_body,
        grid=(1,),
        in_specs=[
            pl.BlockSpec((NC, GQ, 128), lambda i: (0, 0, 0)),
            pl.BlockSpec((NC, GQ, 128), lambda i: (0, 0, 0)),
            pl.BlockSpec((8, 128), lambda i: (0, 0)),
        ],
        out_specs=[
            pl.BlockSpec((GQ, 16, 128), lambda i: (0, 0, 0)),
            pl.BlockSpec((GQ, 16, 128), lambda i: (0, 0, 0)),
        ],
        out_shape=[
            jax.ShapeDtypeStruct((GQ, 16, 128), jnp.float32),
            jax.ShapeDtypeStruct((GQ, 16, 128), jnp.float32),
        ],
    )(dego, degi, e8)


def _prep_body(feat, doutx, tab):
    tab[...] = feat[...] * doutx[...]


def _prep_call(featp, doutx):
    return pl.pallas_call(
        _prep_body,
        grid=(NB,),
        in_specs=[
            pl.BlockSpec((BP, 128), lambda i: (i, 0)),
            pl.BlockSpec((BP, 128), lambda i: (i, 0)),
        ],
        out_specs=pl.BlockSpec((BP, 128), lambda i: (i, 0)),
        out_shape=jax.ShapeDtypeStruct((NP8, 128), jnp.float32),
    )(featp, doutx)


def _bdot(x, w_bf):
    return jnp.dot(x.astype(jnp.bfloat16), w_bf,
                   preferred_element_type=jnp.float32)


def _mid_body(agg, dinx, doutx, bdw1, b1t, tab2):
    t = (agg[0] + agg[1]) * dinx[...]
    h1 = jnp.maximum(_bdot(t, bdw1[...]) + b1t[...], 0.0)
    tab2[...] = h1 * doutx[...]


def _mid_call(agg1, dinx, doutx, bdw1, b1t):
    return pl.pallas_call(
        _mid_body,
        grid=(NB,),
        in_specs=[
            pl.BlockSpec((NC, BP, 128), lambda i: (0, i, 0)),
            pl.BlockSpec((BP, 128), lambda i: (i, 0)),
            pl.BlockSpec((BP, 128), lambda i: (i, 0)),
            pl.BlockSpec((128, 128), lambda i: (0, 0)),
            pl.BlockSpec((1, 128), lambda i: (0, 0)),
        ],
        out_specs=pl.BlockSpec((BP, 128), lambda i: (i, 0)),
        out_shape=jax.ShapeDtypeStruct((NP8, 128), jnp.float32),
    )(agg1, dinx, doutx, bdw1, b1t)


def _fin_body(agg, dinx, bdw2, b2t, bdwr, bdwz, bdwn, brt, bzt, bn1t, bhnt,
              bdwfc, bfct, out):
    t = (agg[0] + agg[1]) * dinx[...]
    h2 = jnp.maximum(_bdot(t, bdw2[...]) + b2t[...], 0.0)
    h2b = h2.astype(jnp.bfloat16)
    r = jax.nn.sigmoid(jnp.dot(h2b, bdwr[...],
                               preferred_element_type=jnp.float32) + brt[...])
    z = jax.nn.sigmoid(jnp.dot(h2b, bdwz[...],
                               preferred_element_type=jnp.float32) + bzt[...])
    n = jnp.tanh(jnp.dot(h2b, bdwn[...], preferred_element_type=jnp.float32)
                 + bn1t[...] + r * bhnt[...])
    hy = (1.0 - z) * n
    out[...] = _bdot(hy, bdwfc[...]) + bfct[...]


def _fin_call(agg2, dinx, bdw2, b2t, bdwr, bdwz, bdwn, brt, bzt, bn1t, bhnt,
              bdwfc, bfct):
    small = lambda r, c: pl.BlockSpec((r, c), lambda i: (0, 0))
    return pl.pallas_call(
        _fin_body,
        grid=(NB,),
        in_specs=[
            pl.BlockSpec((NC, BP, 128), lambda i: (0, i, 0)),
            pl.BlockSpec((BP, 128), lambda i: (i, 0)),
            small(128, 128), small(1, 128),
            small(128, 128), small(128, 128), small(128, 128),
            small(1, 128), small(1, 128), small(1, 128), small(1, 128),
            small(128, 128), small(1, 128),
        ],
        out_specs=pl.BlockSpec((BP, 128), lambda i: (i, 0)),
        out_shape=jax.ShapeDtypeStruct((NP8, 128), jnp.float32),
    )(agg2, dinx, bdw2, b2t, bdwr, bdwz, bdwn, brt, bzt, bn1t, bhnt,
      bdwfc, bfct)


def kernel(features, edge_index, W1, b1, W2, b2, W_ih, b_ih, W_hh, b_hh,
           W_fc, b_fc):
    src = edge_index[0]
    dst = edge_index[1]
    npad = E_PAD - E
    i = jnp.arange(npad, dtype=jnp.int32)
    pad_src = (N + (i % (N_ACC - N))).reshape(npad // CH, CH)
    pad_dst = (N + ((i * 7 + 3) % (N_ACC - N))).reshape(npad // CH, CH)
    src_p = jnp.concatenate([src.reshape(E // CH, CH), pad_src], axis=0)
    dst_p = jnp.concatenate([dst.reshape(E // CH, CH), pad_dst], axis=0)

    dego, degi = _deg_call(src_p, dst_p)

    eye8 = jnp.eye(8, dtype=jnp.float32)
    e8 = jnp.kron(eye8, jnp.ones((1, 16), jnp.float32))
    doutx3, dinx3 = _norm_call(dego.reshape(NC, GQ, 128),
                               degi.reshape(NC, GQ, 128), e8)
    doutx = doutx3.reshape(NP8, 128)
    dinx = dinx3.reshape(NP8, 128)

    featp = jnp.pad(features, ((0, N_ACC - N), (0, H - IN_F))) \
        .reshape(NP8, 128)
    tab1 = _prep_call(featp, doutx)

    agg1 = _agg_call(src_p, dst_p, tab1.reshape(N_ACC, H))

    bf = jnp.bfloat16
    w1p = jnp.zeros((H, H), jnp.float32).at[:IN_F].set(W1)
    tab2 = _mid_call(agg1.reshape(NC, NP8, 128), dinx, doutx,
                     jnp.kron(eye8, w1p).astype(bf),
                     jnp.tile(b1, 8).reshape(1, 128))

    agg2 = _agg_call(src_p, dst_p, tab2.reshape(N_ACC, H))

    wfcp = jnp.zeros((H, H), jnp.float32).at[:, :1].set(W_fc.T)
    outp = _fin_call(
        agg2.reshape(NC, NP8, 128), dinx,
        jnp.kron(eye8, W2).astype(bf),
        jnp.tile(b2, 8).reshape(1, 128),
        jnp.kron(eye8, W_ih[:H].T).astype(bf),
        jnp.kron(eye8, W_ih[H:2 * H].T).astype(bf),
        jnp.kron(eye8, W_ih[2 * H:].T).astype(bf),
        jnp.tile(b_ih[:H] + b_hh[:H], 8).reshape(1, 128),
        jnp.tile(b_ih[H:2 * H] + b_hh[H:2 * H], 8).reshape(1, 128),
        jnp.tile(b_ih[2 * H:], 8).reshape(1, 128),
        jnp.tile(b_hh[2 * H:], 8).reshape(1, 128),
        jnp.kron(eye8, wfcp).astype(bf),
        jnp.tile(jnp.concatenate([b_fc, jnp.zeros((15,), jnp.float32)]),
                 8).reshape(1, 128),
    )
    return outp.reshape(N_ACC, H)[:N, :1]


# speedup vs baseline: 58.9813x; 1.0124x over previous
import functools

import jax
import jax.numpy as jnp
from jax import lax
from jax.experimental import pallas as pl
from jax.experimental.pallas import tpu as pltpu
from jax.experimental.pallas import tpu_sc as plsc

N = 100000
E = 3200000
IN_F = 3
H = 16

NC = 2
NS = 16
NW = NC * NS

N_ACC = 100352
R_T = N_ACC // NS
ZR = 392

CH = 128
SUP = 40
NSUP = 20
CH_PER_TILE = SUP * NSUP
NCHUNK = NW * CH_PER_TILE
E_PAD = NCHUNK * CH

NP8 = N_ACC // 8
GQ = N_ACC // 128
NB = 16
BP = NP8 // NB
BG = GQ // NB

_mesh = plsc.VectorSubcoreMesh(core_axis_name="c", subcore_axis_name="s",
                               num_cores=NC, num_subcores=NS)
_sc_params = pltpu.CompilerParams(use_tc_tiling_on_sc=False)


@functools.partial(
    pl.kernel,
    out_type=[jax.ShapeDtypeStruct((NC, N_ACC), jnp.float32),
              jax.ShapeDtypeStruct((NC, N_ACC), jnp.float32)],
    mesh=_mesh,
    compiler_params=_sc_params,
    scratch_types=[
        pltpu.VMEM((SUP, CH), jnp.int32),
        pltpu.VMEM((SUP, CH), jnp.int32),
        pltpu.VMEM((CH,), jnp.float32),
        pltpu.VMEM((R_T,), jnp.float32),
        pltpu.VMEM_SHARED((N_ACC,), jnp.float32),
        pltpu.VMEM_SHARED((N_ACC,), jnp.float32),
        pltpu.SemaphoreType.DMA,
    ],
)
def _deg_call(src_m, dst_m, src_pd, dst_pd, out_o, out_i, sbuf, dbuf,
              ones_v, zb, acc_o, acc_i, dsem):
    cid = lax.axis_index("c")
    sid = lax.axis_index("s")

    def init_ones(i, _):
        ones_v[pl.ds(i * 16, 16)] = jnp.ones((16,), jnp.float32)
        return 0
    lax.fori_loop(0, CH // 16, init_ones, 0)

    def init_z(i, _):
        zb[pl.ds(i * 16, 16)] = jnp.zeros((16,), jnp.float32)
        return 0
    lax.fori_loop(0, R_T // 16, init_z, 0)

    pltpu.sync_copy(zb, acc_o.at[pl.ds(sid * R_T, R_T)])
    pltpu.sync_copy(zb, acc_i.at[pl.ds(sid * R_T, R_T)])
    plsc.subcore_barrier()

    base_sup = (cid * NS + sid) * NSUP
    SUPM = E // (SUP * CH)

    def super_body(s, _):
        g = base_sup + s

        @pl.when(g < SUPM)
        def _():
            pltpu.sync_copy(src_m.at[pl.ds(g * SUP, SUP)], sbuf)
            pltpu.sync_copy(dst_m.at[pl.ds(g * SUP, SUP)], dbuf)

        @pl.when(g >= SUPM)
        def _():
            pltpu.sync_copy(src_pd.at[pl.ds((g - SUPM) * SUP, SUP)], sbuf)
            pltpu.sync_copy(dst_pd.at[pl.ds((g - SUPM) * SUP, SUP)], dbuf)

        def grp_body(t, _):
            descs = []
            for u in range(8):
                descs.append(pltpu.async_copy(
                    ones_v, acc_o.at[sbuf.at[8 * t + u]], dsem, add=True))
                descs.append(pltpu.async_copy(
                    ones_v, acc_i.at[dbuf.at[8 * t + u]], dsem, add=True))
            for d in descs:
                d.wait()
            return 0
        lax.fori_loop(0, SUP // 8, grp_body, 0)
        return 0
    lax.fori_loop(0, NSUP, super_body, 0)

    plsc.subcore_barrier()
    pltpu.sync_copy(acc_o.at[pl.ds(sid * R_T, R_T)],
                    out_o.at[cid, pl.ds(sid * R_T, R_T)])
    pltpu.sync_copy(acc_i.at[pl.ds(sid * R_T, R_T)],
                    out_i.at[cid, pl.ds(sid * R_T, R_T)])


@functools.partial(
    pl.kernel,
    out_type=jax.ShapeDtypeStruct((NC, N_ACC, H), jnp.float32),
    mesh=_mesh,
    compiler_params=_sc_params,
    scratch_types=[
        pltpu.VMEM((SUP, CH), jnp.int32),
        pltpu.VMEM((SUP, CH), jnp.int32),
        pltpu.VMEM((5 * CH, H), jnp.float32),
        pltpu.VMEM((5 * CH, H), jnp.float32),
        pltpu.VMEM_SHARED((N_ACC, H), jnp.float32),
        pltpu.SemaphoreType.DMA,
        pltpu.SemaphoreType.DMA,
    ],
)
def _agg_call(src_hbm, dst_hbm, tab_hbm, out, sbuf, dbuf, rows_a, rows_b,
              acc, gsa, gsb):
    cid = lax.axis_index("c")
    sid = lax.axis_index("s")

    def init_z(i, _):
        rows_a[i] = jnp.zeros((H,), jnp.float32)
        return 0
    lax.fori_loop(0, 5 * CH, init_z, 0)

    def zero_acc(k, _):
        pltpu.sync_copy(rows_a, acc.at[pl.ds(sid * R_T + k * 5 * CH, 5 * CH)])
        return 0
    lax.fori_loop(0, R_T // (5 * CH), zero_acc, 0)
    pltpu.sync_copy(rows_a.at[pl.ds(0, 4 * CH)],
                    acc.at[pl.ds(sid * R_T + (R_T // (5 * CH)) * 5 * CH,
                                 4 * CH)])
    plsc.subcore_barrier()

    base_chunk = (cid * NS + sid) * CH_PER_TILE

    def super_body(s, _):
        pltpu.sync_copy(src_hbm.at[pl.ds(base_chunk + s * SUP, SUP)], sbuf)
        pltpu.sync_copy(dst_hbm.at[pl.ds(base_chunk + s * SUP, SUP)], dbuf)

        G = 5
        for u in range(G):
            pltpu.async_copy(tab_hbm.at[sbuf.at[u]],
                             rows_a.at[pl.ds(CH * u, CH)], gsa)

        def grp_pair(t, _):
            b0 = 2 * G * t
            for u in range(G):
                pltpu.async_copy(tab_hbm.at[sbuf.at[b0 + G + u]],
                                 rows_b.at[pl.ds(CH * u, CH)], gsb)
            for u in range(G):
                pltpu.make_async_copy(tab_hbm.at[sbuf.at[b0 + u]],
                                      rows_a.at[pl.ds(CH * u, CH)],
                                      gsa).wait()
            for u in range(G):
                pltpu.sync_copy(rows_a.at[pl.ds(CH * u, CH)],
                                acc.at[dbuf.at[b0 + u]], add=True)

            @pl.when(t < SUP // (2 * G) - 1)
            def _():
                for u in range(G):
                    pltpu.async_copy(tab_hbm.at[sbuf.at[b0 + 2 * G + u]],
                                     rows_a.at[pl.ds(CH * u, CH)], gsa)
            for u in range(G):
                pltpu.make_async_copy(tab_hbm.at[sbuf.at[b0 + G + u]],
                                      rows_b.at[pl.ds(CH * u, CH)],
                                      gsb).wait()
            for u in range(G):
                pltpu.sync_copy(rows_b.at[pl.ds(CH * u, CH)],
                                acc.at[dbuf.at[b0 + G + u]], add=True)
            return 0
        lax.fori_loop(0, SUP // (2 * G), grp_pair, 0)
        return 0
    lax.fori_loop(0, NSUP, super_body, 0)

    plsc.subcore_barrier()
    pltpu.sync_copy(acc.at[pl.ds(sid * R_T, R_T)],
                    out.at[cid, pl.ds(sid * R_T, R_T)])


def _norm_body(dego, degi, e8, doutx, dinx):
    dn_o = lax.rsqrt(jnp.maximum(dego[0] + dego[1], 1.0))
    dn_i = lax.rsqrt(jnp.maximum(degi[0] + degi[1], 1.0))
    for s in range(16):
        doutx[:, s, :] = jnp.dot(dn_o[:, 8 * s:8 * s + 8], e8[...],
                                 preferred_element_type=jnp.float32,
                       precision=lax.Precision.HIGHEST)
        dinx[:, s, :] = jnp.dot(dn_i[:, 8 * s:8 * s + 8], e8[...],
                                preferred_element_type=jnp.float32,
                       precision=lax.Precision.HIGHEST)


def _norm_call(dego, degi, e8):
    return pl.pallas_call(
        _norm_body,
        grid=(1,),
        in_specs=[
            pl.BlockSpec((NC, GQ, 128), lambda i: (0, 0, 0)),
            pl.BlockSpec((NC, GQ, 128), lambda i: (0, 0, 0)),
            pl.BlockSpec((8, 128), lambda i: (0, 0)),
        ],
        out_specs=[
            pl.BlockSpec((GQ, 16, 128), lambda i: (0, 0, 0)),
            pl.BlockSpec((GQ, 16, 128), lambda i: (0, 0, 0)),
        ],
        out_shape=[
            jax.ShapeDtypeStruct((GQ, 16, 128), jnp.float32),
            jax.ShapeDtypeStruct((GQ, 16, 128), jnp.float32),
        ],
    )(dego, degi, e8)


def _prep_body(feat, doutx, tab):
    tab[...] = feat[...] * doutx[...]


def _prep_call(featp, doutx):
    return pl.pallas_call(
        _prep_body,
        grid=(NB,),
        in_specs=[
            pl.BlockSpec((BP, 128), lambda i: (i, 0)),
            pl.BlockSpec((BP, 128), lambda i: (i, 0)),
        ],
        out_specs=pl.BlockSpec((BP, 128), lambda i: (i, 0)),
        out_shape=jax.ShapeDtypeStruct((NP8, 128), jnp.float32),
    )(featp, doutx)


def _bdot(x, w_bf):
    return jnp.dot(x.astype(jnp.bfloat16), w_bf,
                   preferred_element_type=jnp.float32)


def _mid_body(agg, dinx, doutx, bdw1, b1t, tab2):
    t = (agg[0] + agg[1]) * dinx[...]
    h1 = jnp.maximum(_bdot(t, bdw1[...]) + b1t[...], 0.0)
    tab2[...] = h1 * doutx[...]


def _mid_call(agg1, dinx, doutx, bdw1, b1t):
    return pl.pallas_call(
        _mid_body,
        grid=(NB,),
        in_specs=[
            pl.BlockSpec((NC, BP, 128), lambda i: (0, i, 0)),
            pl.BlockSpec((BP, 128), lambda i: (i, 0)),
            pl.BlockSpec((BP, 128), lambda i: (i, 0)),
            pl.BlockSpec((128, 128), lambda i: (0, 0)),
            pl.BlockSpec((1, 128), lambda i: (0, 0)),
        ],
        out_specs=pl.BlockSpec((BP, 128), lambda i: (i, 0)),
        out_shape=jax.ShapeDtypeStruct((NP8, 128), jnp.float32),
    )(agg1, dinx, doutx, bdw1, b1t)


def _fin_body(agg, dinx, bdw2, b2t, bdwr, bdwz, bdwn, brt, bzt, bn1t, bhnt,
              bdwfc, bfct, out):
    t = (agg[0] + agg[1]) * dinx[...]
    h2 = jnp.maximum(_bdot(t, bdw2[...]) + b2t[...], 0.0)
    h2b = h2.astype(jnp.bfloat16)
    r = jax.nn.sigmoid(jnp.dot(h2b, bdwr[...],
                               preferred_element_type=jnp.float32) + brt[...])
    z = jax.nn.sigmoid(jnp.dot(h2b, bdwz[...],
                               preferred_element_type=jnp.float32) + bzt[...])
    n = jnp.tanh(jnp.dot(h2b, bdwn[...], preferred_element_type=jnp.float32)
                 + bn1t[...] + r * bhnt[...])
    hy = (1.0 - z) * n
    out[...] = _bdot(hy, bdwfc[...]) + bfct[...]


def _fin_call(agg2, dinx, bdw2, b2t, bdwr, bdwz, bdwn, brt, bzt, bn1t, bhnt,
              bdwfc, bfct):
    small = lambda r, c: pl.BlockSpec((r, c), lambda i: (0, 0))
    return pl.pallas_call(
        _fin_body,
        grid=(NB,),
        in_specs=[
            pl.BlockSpec((NC, BP, 128), lambda i: (0, i, 0)),
            pl.BlockSpec((BP, 128), lambda i: (i, 0)),
            small(128, 128), small(1, 128),
            small(128, 128), small(128, 128), small(128, 128),
            small(1, 128), small(1, 128), small(1, 128), small(1, 128),
            small(128, 128), small(1, 128),
        ],
        out_specs=pl.BlockSpec((BP, 128), lambda i: (i, 0)),
        out_shape=jax.ShapeDtypeStruct((NP8, 128), jnp.float32),
    )(agg2, dinx, bdw2, b2t, bdwr, bdwz, bdwn, brt, bzt, bn1t, bhnt,
      bdwfc, bfct)


def kernel(features, edge_index, W1, b1, W2, b2, W_ih, b_ih, W_hh, b_hh,
           W_fc, b_fc):
    src = edge_index[0]
    dst = edge_index[1]
    npad = E_PAD - E
    i = jnp.arange(npad, dtype=jnp.int32)
    pad_src = (N + (i % (N_ACC - N))).reshape(npad // CH, CH)
    pad_dst = (N + ((i * 7 + 3) % (N_ACC - N))).reshape(npad // CH, CH)
    src_p = jnp.concatenate([src.reshape(E // CH, CH), pad_src], axis=0)
    dst_p = jnp.concatenate([dst.reshape(E // CH, CH), pad_dst], axis=0)

    dego, degi = _deg_call(src.reshape(E // CH, CH), dst.reshape(E // CH, CH),
                           pad_src, pad_dst)

    eye8 = jnp.eye(8, dtype=jnp.float32)
    e8 = jnp.kron(eye8, jnp.ones((1, 16), jnp.float32))
    doutx3, dinx3 = _norm_call(dego.reshape(NC, GQ, 128),
                               degi.reshape(NC, GQ, 128), e8)
    doutx = doutx3.reshape(NP8, 128)
    dinx = dinx3.reshape(NP8, 128)

    featp = jnp.pad(features, ((0, N_ACC - N), (0, H - IN_F))) \
        .reshape(NP8, 128)
    tab1 = _prep_call(featp, doutx)

    agg1 = _agg_call(src_p, dst_p, tab1.reshape(N_ACC, H))

    bf = jnp.bfloat16
    w1p = jnp.zeros((H, H), jnp.float32).at[:IN_F].set(W1)
    tab2 = _mid_call(agg1.reshape(NC, NP8, 128), dinx, doutx,
                     jnp.kron(eye8, w1p).astype(bf),
                     jnp.tile(b1, 8).reshape(1, 128))

    agg2 = _agg_call(src_p, dst_p, tab2.reshape(N_ACC, H))

    wfcp = jnp.zeros((H, H), jnp.float32).at[:, :1].set(W_fc.T)
    outp = _fin_call(
        agg2.reshape(NC, NP8, 128), dinx,
        jnp.kron(eye8, W2).astype(bf),
        jnp.tile(b2, 8).reshape(1, 128),
        jnp.kron(eye8, W_ih[:H].T).astype(bf),
        jnp.kron(eye8, W_ih[H:2 * H].T).astype(bf),
        jnp.kron(eye8, W_ih[2 * H:].T).astype(bf),
        jnp.tile(b_ih[:H] + b_hh[:H], 8).reshape(1, 128),
        jnp.tile(b_ih[H:2 * H] + b_hh[H:2 * H], 8).reshape(1, 128),
        jnp.tile(b_ih[2 * H:], 8).reshape(1, 128),
        jnp.tile(b_hh[2 * H:], 8).reshape(1, 128),
        jnp.kron(eye8, wfcp).astype(bf),
        jnp.tile(jnp.concatenate([b_fc, jnp.zeros((15,), jnp.float32)]),
                 8).reshape(1, 128),
    )
    return outp.reshape(N_ACC, H)[:N, :1]
